# baseline (speedup 1.0000x reference)
import functools

import jax
import jax.numpy as jnp
from jax import lax
from jax.experimental import pallas as pl
from jax.experimental.pallas import tpu as pltpu

HIDDEN = 768
FFN = 1536
E = 64
TOP_K = 2
TOKENS = 2048
PAIRS = TOKENS * TOP_K
CHUNK = 64


def _router_body(x_ref, gw_ref, logits_ref, sel_ref, wts_ref):
    x = x_ref[...]
    gw = gw_ref[...]
    logits = lax.dot_general(
        x, gw, (((1,), (1,)), ((), ())), preferred_element_type=jnp.float32
    )
    logits_ref[...] = logits
    iota = lax.broadcasted_iota(jnp.int32, logits.shape, 1)
    m1 = jnp.max(logits, axis=1, keepdims=True)
    a1 = jnp.min(jnp.where(logits == m1, iota, E), axis=1, keepdims=True)
    neg = jnp.full_like(logits, -jnp.inf)
    l2 = jnp.where(iota == a1, neg, logits)
    m2 = jnp.max(l2, axis=1, keepdims=True)
    a2 = jnp.min(jnp.where(l2 == m2, iota, E), axis=1, keepdims=True)
    e2 = jnp.exp(m2 - m1)
    w1v = 1.0 / (1.0 + e2)
    w2v = e2 / (1.0 + e2)
    sel_ref[...] = jnp.concatenate([a1.T, a2.T], axis=0)
    wts_ref[...] = jnp.concatenate([w1v.T, w2v.T], axis=0)


def _moe_body(tok_ref, off_ref, w_ref, x_ref, w1_ref, w2_ref, w3_ref,
              out_ref, xg_ref, h_ref):
    e = pl.program_id(0)

    @pl.when(e == 0)
    def _():
        out_ref[...] = jnp.zeros_like(out_ref)

    start = off_ref[e]
    end = off_ref[e + 1]
    count = end - start
    nchunks = (count + CHUNK - 1) // CHUNK

    def chunk_body(c, _):
        base = start + c * CHUNK

        def gather_row(r, _):
            idx = jnp.minimum(base + r, PAIRS - 1)
            tok = tok_ref[idx]
            xg_ref[pl.ds(r, 1), :] = x_ref[pl.ds(tok, 1), :]
            return 0

        lax.fori_loop(0, CHUNK, gather_row, 0, unroll=8)

        xg = xg_ref[...].astype(jnp.bfloat16)
        a = lax.dot_general(xg, w1_ref[0].astype(jnp.bfloat16),
                            (((1,), (1,)), ((), ())),
                            preferred_element_type=jnp.float32)
        b = lax.dot_general(xg, w3_ref[0].astype(jnp.bfloat16),
                            (((1,), (1,)), ((), ())),
                            preferred_element_type=jnp.float32)
        g = (a * jax.nn.sigmoid(a) * b).astype(jnp.bfloat16)
        h_ref[...] = lax.dot_general(g, w2_ref[0].astype(jnp.bfloat16),
                                     (((1,), (1,)), ((), ())),
                                     preferred_element_type=jnp.float32)

        def scatter_row(r, _):
            idx = base + r
            idc = jnp.minimum(idx, PAIRS - 1)
            tok = tok_ref[idc]
            w = jnp.where(idx < end, w_ref[idc], 0.0)
            out_ref[pl.ds(tok, 1), :] += h_ref[pl.ds(r, 1), :] * w
            return 0

        lax.fori_loop(0, CHUNK, scatter_row, 0, unroll=8)
        return 0

    lax.fori_loop(0, nchunks, chunk_body, 0)


@jax.jit
def kernel(hidden_states, gate_w, w1, w2, w3):
    B, S, H = hidden_states.shape
    x = hidden_states.reshape(S, H)

    logits, sel, wts = pl.pallas_call(
        _router_body,
        out_shape=[
            jax.ShapeDtypeStruct((S, E), jnp.float32),
            jax.ShapeDtypeStruct((TOP_K, S), jnp.int32),
            jax.ShapeDtypeStruct((TOP_K, S), jnp.float32),
        ],
    )(x, gate_w)

    e_flat = sel.reshape(-1)
    order = jnp.argsort(e_flat)
    tok_sorted = (order % S).astype(jnp.int32)
    w_sorted = wts.reshape(-1)[order]
    counts = jnp.bincount(e_flat, length=E)
    offsets = jnp.concatenate(
        [jnp.zeros((1,), jnp.int32), jnp.cumsum(counts).astype(jnp.int32)]
    )

    out = pl.pallas_call(
        _moe_body,
        grid=(E,),
        in_specs=[
            pl.BlockSpec(memory_space=pltpu.SMEM),
            pl.BlockSpec(memory_space=pltpu.SMEM),
            pl.BlockSpec(memory_space=pltpu.SMEM),
            pl.BlockSpec((S, H), lambda e: (0, 0)),
            pl.BlockSpec((1, FFN, H), lambda e: (e, 0, 0)),
            pl.BlockSpec((1, H, FFN), lambda e: (e, 0, 0)),
            pl.BlockSpec((1, FFN, H), lambda e: (e, 0, 0)),
        ],
        out_specs=pl.BlockSpec((S, H), lambda e: (0, 0)),
        out_shape=jax.ShapeDtypeStruct((S, H), jnp.float32),
        scratch_shapes=[
            pltpu.VMEM((CHUNK, H), jnp.float32),
            pltpu.VMEM((CHUNK, H), jnp.float32),
        ],
        compiler_params=pltpu.CompilerParams(
            dimension_semantics=("arbitrary",),
        ),
    )(tok_sorted, offsets, w_sorted, x, w1, w2, w3)

    return out.reshape(B, S, H), logits


# speedup vs baseline: 1.1844x; 1.1844x over previous
import functools

import jax
import jax.numpy as jnp
from jax import lax
from jax.experimental import pallas as pl
from jax.experimental.pallas import tpu as pltpu

HIDDEN = 768
FFN = 1536
E = 64
TOP_K = 2
TOKENS = 2048
PAIRS = TOKENS * TOP_K
CHUNK = 128


def _router_body(x_ref, gw_ref, logits_ref, sel_ref, wts_ref):
    x = x_ref[...]
    gw = gw_ref[...]
    logits = lax.dot_general(
        x, gw, (((1,), (1,)), ((), ())), preferred_element_type=jnp.float32
    )
    logits_ref[...] = logits
    iota = lax.broadcasted_iota(jnp.int32, logits.shape, 1)
    m1 = jnp.max(logits, axis=1, keepdims=True)
    a1 = jnp.min(jnp.where(logits == m1, iota, E), axis=1, keepdims=True)
    neg = jnp.full_like(logits, -jnp.inf)
    l2 = jnp.where(iota == a1, neg, logits)
    m2 = jnp.max(l2, axis=1, keepdims=True)
    a2 = jnp.min(jnp.where(l2 == m2, iota, E), axis=1, keepdims=True)
    e2 = jnp.exp(m2 - m1)
    w1v = 1.0 / (1.0 + e2)
    w2v = e2 / (1.0 + e2)
    sel_ref[...] = jnp.concatenate([a1.T, a2.T], axis=0)
    wts_ref[...] = jnp.concatenate([w1v.T, w2v.T], axis=0)


def _moe_body(tok_ref, off_ref, w_ref, x_ref, w1_ref, w2_ref, w3_ref,
              out_ref, xg_ref, h_ref, wc_ref):
    e = pl.program_id(0)

    @pl.when(e == 0)
    def _():
        out_ref[...] = jnp.zeros_like(out_ref)

    start = off_ref[e]
    end = off_ref[e + 1]
    count = end - start
    nchunks = (count + CHUNK - 1) // CHUNK

    def chunk_body(c, _):
        base = start + c * CHUNK
        rem = jnp.minimum(end - base, CHUNK)
        ngroups = (rem + 7) // 8

        def gather_group(i, _):
            r0 = i * 8
            for j in range(8):
                r = r0 + j
                idx = jnp.minimum(base + r, end - 1)
                tok = tok_ref[idx]
                xg_ref[pl.ds(r, 1), :] = x_ref[pl.ds(tok, 1), :]
                w = jnp.where(base + r < end, w_ref[idx], 0.0)
                wc_ref[pl.ds(r, 1), :] = jnp.full((1, 1), w, jnp.float32)
            return 0

        lax.fori_loop(0, ngroups, gather_group, 0)

        xg = xg_ref[...].astype(jnp.bfloat16)
        a = lax.dot_general(xg, w1_ref[0].astype(jnp.bfloat16),
                            (((1,), (1,)), ((), ())),
                            preferred_element_type=jnp.float32)
        b = lax.dot_general(xg, w3_ref[0].astype(jnp.bfloat16),
                            (((1,), (1,)), ((), ())),
                            preferred_element_type=jnp.float32)
        g = (a * jax.nn.sigmoid(a) * b).astype(jnp.bfloat16)
        h = lax.dot_general(g, w2_ref[0].astype(jnp.bfloat16),
                            (((1,), (1,)), ((), ())),
                            preferred_element_type=jnp.float32)
        h_ref[...] = h * wc_ref[...]

        def scatter_group(i, _):
            r0 = i * 8
            for j in range(8):
                r = r0 + j
                idx = jnp.minimum(base + r, end - 1)
                tok = tok_ref[idx]
                out_ref[pl.ds(tok, 1), :] += h_ref[pl.ds(r, 1), :]
            return 0

        lax.fori_loop(0, ngroups, scatter_group, 0)
        return 0

    lax.fori_loop(0, nchunks, chunk_body, 0)


@jax.jit
def kernel(hidden_states, gate_w, w1, w2, w3):
    B, S, H = hidden_states.shape
    x = hidden_states.reshape(S, H)

    logits, sel, wts = pl.pallas_call(
        _router_body,
        out_shape=[
            jax.ShapeDtypeStruct((S, E), jnp.float32),
            jax.ShapeDtypeStruct((TOP_K, S), jnp.int32),
            jax.ShapeDtypeStruct((TOP_K, S), jnp.float32),
        ],
    )(x, gate_w)

    e_flat = sel.reshape(-1)
    order = jnp.argsort(e_flat)
    tok_sorted = (order % S).astype(jnp.int32)
    w_sorted = wts.reshape(-1)[order]
    counts = jnp.bincount(e_flat, length=E)
    offsets = jnp.concatenate(
        [jnp.zeros((1,), jnp.int32), jnp.cumsum(counts).astype(jnp.int32)]
    )

    out = pl.pallas_call(
        _moe_body,
        grid=(E,),
        in_specs=[
            pl.BlockSpec(memory_space=pltpu.SMEM),
            pl.BlockSpec(memory_space=pltpu.SMEM),
            pl.BlockSpec(memory_space=pltpu.SMEM),
            pl.BlockSpec((S, H), lambda e: (0, 0)),
            pl.BlockSpec((1, FFN, H), lambda e: (e, 0, 0)),
            pl.BlockSpec((1, H, FFN), lambda e: (e, 0, 0)),
            pl.BlockSpec((1, FFN, H), lambda e: (e, 0, 0)),
        ],
        out_specs=pl.BlockSpec((S, H), lambda e: (0, 0)),
        out_shape=jax.ShapeDtypeStruct((S, H), jnp.float32),
        scratch_shapes=[
            pltpu.VMEM((CHUNK, H), jnp.float32),
            pltpu.VMEM((CHUNK, H), jnp.float32),
            pltpu.VMEM((CHUNK, 1), jnp.float32),
        ],
        compiler_params=pltpu.CompilerParams(
            dimension_semantics=("arbitrary",),
        ),
    )(tok_sorted, offsets, w_sorted, x, w1, w2, w3)

    return out.reshape(B, S, H), logits


# speedup vs baseline: 1.2798x; 1.0806x over previous
import functools

import jax
import jax.numpy as jnp
from jax import lax
from jax.experimental import pallas as pl
from jax.experimental.pallas import tpu as pltpu

HIDDEN = 768
FFN = 1536
E = 64
TOP_K = 2
TOKENS = 2048
PAIRS = TOKENS * TOP_K
CHUNK = 128


def _router_body(x_ref, gw_ref, logits_ref, sel_ref, wts_ref):
    x = x_ref[...]
    gw = gw_ref[...]
    logits = lax.dot_general(
        x, gw, (((1,), (1,)), ((), ())), preferred_element_type=jnp.float32
    )
    logits_ref[...] = logits
    iota = lax.broadcasted_iota(jnp.int32, logits.shape, 1)
    m1 = jnp.max(logits, axis=1, keepdims=True)
    a1 = jnp.min(jnp.where(logits == m1, iota, E), axis=1, keepdims=True)
    neg = jnp.full_like(logits, -jnp.inf)
    l2 = jnp.where(iota == a1, neg, logits)
    m2 = jnp.max(l2, axis=1, keepdims=True)
    a2 = jnp.min(jnp.where(l2 == m2, iota, E), axis=1, keepdims=True)
    e2 = jnp.exp(m2 - m1)
    w1v = 1.0 / (1.0 + e2)
    w2v = e2 / (1.0 + e2)
    sel_ref[...] = jnp.concatenate([a1.T, a2.T], axis=0)
    wts_ref[...] = jnp.concatenate([w1v.T, w2v.T], axis=0)


def _moe_body(tok_ref, off_ref, w_ref, x_ref, w1_ref, w2_ref, w3_ref,
              out_ref, xg_ref, h_ref, wc_ref):
    e = pl.program_id(0)

    @pl.when(e == 0)
    def _():
        out_ref[...] = jnp.zeros_like(out_ref)

    start = off_ref[e]
    end = off_ref[e + 1]
    count = end - start
    nchunks = (count + CHUNK - 1) // CHUNK

    def chunk_body(c, _):
        base = start + c * CHUNK
        rem = jnp.minimum(end - base, CHUNK)
        ngroups = (rem + 15) // 16

        def gather_group(i, _):
            r0 = i * 16
            for j in range(16):
                r = r0 + j
                idx = jnp.minimum(base + r, end - 1)
                pair = tok_ref[idx]
                tok = pair & (TOKENS - 1)
                xg_ref[pl.ds(r, 1), :] = x_ref[pl.ds(tok, 1), :]
                w = jnp.where(base + r < end, w_ref[pair], 0.0)
                wc_ref[pl.ds(r, 1), :] = jnp.full((1, 1), w, jnp.float32)
            return 0

        lax.fori_loop(0, ngroups, gather_group, 0)

        xg = xg_ref[...].astype(jnp.bfloat16)
        a = lax.dot_general(xg, w1_ref[0].astype(jnp.bfloat16),
                            (((1,), (1,)), ((), ())),
                            preferred_element_type=jnp.float32)
        b = lax.dot_general(xg, w3_ref[0].astype(jnp.bfloat16),
                            (((1,), (1,)), ((), ())),
                            preferred_element_type=jnp.float32)
        g = (a * jax.nn.sigmoid(a) * b).astype(jnp.bfloat16)
        h = lax.dot_general(g, w2_ref[0].astype(jnp.bfloat16),
                            (((1,), (1,)), ((), ())),
                            preferred_element_type=jnp.float32)
        h_ref[...] = h * wc_ref[...]

        def scatter_group(i, _):
            r0 = i * 16
            for j in range(16):
                r = r0 + j
                idx = jnp.minimum(base + r, end - 1)
                tok = tok_ref[idx] & (TOKENS - 1)
                out_ref[pl.ds(tok, 1), :] += h_ref[pl.ds(r, 1), :]
            return 0

        lax.fori_loop(0, ngroups, scatter_group, 0)
        return 0

    lax.fori_loop(0, nchunks, chunk_body, 0)


@jax.jit
def kernel(hidden_states, gate_w, w1, w2, w3):
    B, S, H = hidden_states.shape
    x = hidden_states.reshape(S, H)

    logits, sel, wts = pl.pallas_call(
        _router_body,
        out_shape=[
            jax.ShapeDtypeStruct((S, E), jnp.float32),
            jax.ShapeDtypeStruct((TOP_K, S), jnp.int32),
            jax.ShapeDtypeStruct((TOP_K, S), jnp.float32),
        ],
    )(x, gate_w)

    e_flat = sel.reshape(-1)
    order = jnp.argsort(e_flat).astype(jnp.int32)
    counts = jnp.sum(
        (e_flat[:, None] == jnp.arange(E, dtype=jnp.int32)[None, :])
        .astype(jnp.int32),
        axis=0,
    )
    offsets = jnp.concatenate(
        [jnp.zeros((1,), jnp.int32), jnp.cumsum(counts).astype(jnp.int32)]
    )
    w_flat = wts.reshape(-1)

    out = pl.pallas_call(
        _moe_body,
        grid=(E,),
        in_specs=[
            pl.BlockSpec(memory_space=pltpu.SMEM),
            pl.BlockSpec(memory_space=pltpu.SMEM),
            pl.BlockSpec(memory_space=pltpu.SMEM),
            pl.BlockSpec((S, H), lambda e: (0, 0)),
            pl.BlockSpec((1, FFN, H), lambda e: (e, 0, 0)),
            pl.BlockSpec((1, H, FFN), lambda e: (e, 0, 0)),
            pl.BlockSpec((1, FFN, H), lambda e: (e, 0, 0)),
        ],
        out_specs=pl.BlockSpec((S, H), lambda e: (0, 0)),
        out_shape=jax.ShapeDtypeStruct((S, H), jnp.float32),
        scratch_shapes=[
            pltpu.VMEM((CHUNK, H), jnp.float32),
            pltpu.VMEM((CHUNK, H), jnp.float32),
            pltpu.VMEM((CHUNK, 1), jnp.float32),
        ],
        compiler_params=pltpu.CompilerParams(
            dimension_semantics=("arbitrary",),
        ),
    )(order, offsets, w_flat, x, w1, w2, w3)

    return out.reshape(B, S, H), logits
